# baseline (speedup 1.0000x reference)
import functools

import jax
import jax.numpy as jnp
from jax import lax
from jax.experimental import pallas as pl
from jax.experimental.pallas import tpu as pltpu
from jax.experimental.pallas import tpu_sc as plsc

B = 16384
F = 512
G = 8
P = F // G
L = 16
NC = 2
NS = 16
NW = NC * NS

F_SC = 256
P_SC = F_SC // G
F_TC = F - F_SC
P_TC = P - P_SC
ROWS_PER_W = B // NW
CH = 64
NCHUNK = ROWS_PER_W // CH
NG = P_SC // L

_mesh = plsc.VectorSubcoreMesh(core_axis_name="c", subcore_axis_name="s")


@functools.partial(
    pl.kernel,
    out_type=jax.ShapeDtypeStruct((P_SC, B), jnp.float32),
    mesh=_mesh,
    scratch_types=[
        pltpu.VMEM((CH, F_SC), jnp.float32),
        pltpu.VMEM((CH, F_SC), jnp.float32),
        pltpu.VMEM((NG * CH * L,), jnp.float32),
        pltpu.VMEM((P_SC, 2 * CH), jnp.float32),
        pltpu.VMEM((P_SC, 2 * CH), jnp.float32),
        pltpu.SemaphoreType.DMA,
        pltpu.SemaphoreType.DMA,
        pltpu.SemaphoreType.DMA,
        pltpu.SemaphoreType.DMA,
    ],
    compiler_params=pltpu.CompilerParams(
        needs_layout_passes=False, skip_device_barrier=True
    ),
)
def _pool_sc(x_hbm, out_hbm, in0, in1, stage, o0, o1, si0, si1, so0, so1):
    wid = lax.axis_index("s") * NC + lax.axis_index("c")
    row0 = wid * ROWS_PER_W

    ins = (in0, in1)
    outs = (o0, o1)
    isems = (si0, si1)
    osems = (so0, so1)

    lane = lax.iota(jnp.int32, L)
    lane8 = lane * G
    lane16 = lane * L

    in_copies = [None, None]
    out_copies = [None, None]
    in_copies[0] = pltpu.async_copy(
        x_hbm.at[pl.ds(row0, CH), pl.ds(0, F_SC)], ins[0], isems[0]
    )

    for c in range(NCHUNK):
        cur = c % 2
        if c + 1 < NCHUNK:
            nxt = (c + 1) % 2
            in_copies[nxt] = pltpu.async_copy(
                x_hbm.at[pl.ds(row0 + (c + 1) * CH, CH), pl.ds(0, F_SC)],
                ins[nxt],
                isems[nxt],
            )
        in_copies[cur].wait()
        ob = (c // 2) % 2
        half = c % 2
        if half == 0 and out_copies[ob] is not None:
            out_copies[ob].wait()

        in_ref = ins[cur]
        out_ref = outs[ob]

        @plsc.parallel_loop(0, CH, step=1, unroll=2)
        def _sums(r):
            row_idx = jnp.full((L,), r, jnp.int32)
            for g in range(NG):
                col0 = lane8 + g * (L * G)
                acc = plsc.load_gather(in_ref, [row_idx, col0])
                for k in range(1, G):
                    acc = acc + plsc.load_gather(in_ref, [row_idx, col0 + k])
                stage[pl.ds((g * CH + r) * L, L)] = acc * (1.0 / G)

        @plsc.parallel_loop(0, P_SC, step=1, unroll=2)
        def _tr(i):
            g = i // L
            col = i % L
            for q in range(CH // L):
                idx = lane16 + ((g * CH + q * L) * L + col)
                v = plsc.load_gather(stage, [idx])
                out_ref[i, pl.ds(half * CH + q * L, L)] = v

        if half == 1:
            out_copies[ob] = pltpu.async_copy(
                out_ref,
                out_hbm.at[:, pl.ds(row0 + (c - 1) * CH, 2 * CH)],
                osems[ob],
            )

    out_copies[0].wait()
    out_copies[1].wait()


RB = 2048


def _pool_tc_body(x_ref, o_ref):
    f = lax.broadcasted_iota(jnp.int32, (P_TC, F_TC), 1)
    p = lax.broadcasted_iota(jnp.int32, (P_TC, F_TC), 0)
    w = jnp.where(f // G == p, 1.0 / G, 0.0).astype(jnp.float32)
    o_ref[...] = lax.dot_general(
        w,
        x_ref[...],
        (((1,), (1,)), ((), ())),
        preferred_element_type=jnp.float32,
        precision=lax.Precision.HIGHEST,
    )


_pool_tc = pl.pallas_call(
    _pool_tc_body,
    grid=(B // RB,),
    in_specs=[pl.BlockSpec((RB, F_TC), lambda i: (i, 1))],
    out_specs=pl.BlockSpec((P_TC, RB), lambda i: (0, i)),
    out_shape=jax.ShapeDtypeStruct((P_TC, B), jnp.float32),
)


def kernel(gene_set_features):
    o_sc = _pool_sc(gene_set_features)
    o_tc = _pool_tc(gene_set_features)
    return jnp.concatenate([o_sc, o_tc], axis=0).T


# speedup vs baseline: 1.0567x; 1.0567x over previous
import functools

import jax
import jax.numpy as jnp
from jax import lax
from jax.experimental import pallas as pl
from jax.experimental.pallas import tpu as pltpu
from jax.experimental.pallas import tpu_sc as plsc

B = 16384
F = 512
G = 8
P = F // G
L = 16
NC = 2
NS = 16
NW = NC * NS

B_SC = 8192
B_TC = B - B_SC
ROWS_PER_W = B_SC // NW
CH = 64
NCHUNK = ROWS_PER_W // CH
NG = P // L

_mesh = plsc.VectorSubcoreMesh(core_axis_name="c", subcore_axis_name="s")


@functools.partial(
    pl.kernel,
    out_type=jax.ShapeDtypeStruct((P, B_SC), jnp.float32),
    mesh=_mesh,
    scratch_types=[
        pltpu.VMEM((CH, F), jnp.float32),
        pltpu.VMEM((CH, F), jnp.float32),
        pltpu.VMEM((NG * CH * L,), jnp.float32),
        pltpu.VMEM((P, 2 * CH), jnp.float32),
        pltpu.VMEM((P, 2 * CH), jnp.float32),
        pltpu.SemaphoreType.DMA,
        pltpu.SemaphoreType.DMA,
        pltpu.SemaphoreType.DMA,
        pltpu.SemaphoreType.DMA,
    ],
    compiler_params=pltpu.CompilerParams(
        needs_layout_passes=False, skip_device_barrier=True
    ),
)
def _pool_sc(x_hbm, out_hbm, in0, in1, stage, o0, o1, si0, si1, so0, so1):
    wid = lax.axis_index("s") * NC + lax.axis_index("c")
    row0 = wid * ROWS_PER_W

    ins = (in0, in1)
    outs = (o0, o1)
    isems = (si0, si1)
    osems = (so0, so1)

    lane = lax.iota(jnp.int32, L)
    lane8 = lane * G
    lane16 = lane * L

    in_copies = [None, None]
    out_copies = [None, None]
    in_copies[0] = pltpu.async_copy(
        x_hbm.at[pl.ds(row0, CH)], ins[0], isems[0]
    )

    for c in range(NCHUNK):
        cur = c % 2
        if c + 1 < NCHUNK:
            nxt = (c + 1) % 2
            in_copies[nxt] = pltpu.async_copy(
                x_hbm.at[pl.ds(row0 + (c + 1) * CH, CH)],
                ins[nxt],
                isems[nxt],
            )
        in_copies[cur].wait()
        ob = (c // 2) % 2
        half = c % 2
        if half == 0 and out_copies[ob] is not None:
            out_copies[ob].wait()

        in_ref = ins[cur]
        out_ref = outs[ob]

        @plsc.parallel_loop(0, CH, step=1, unroll=2)
        def _sums(r):
            row_idx = jnp.full((L,), r, jnp.int32)
            for g in range(NG):
                col0 = lane8 + g * (L * G)
                acc = plsc.load_gather(in_ref, [row_idx, col0])
                for k in range(1, G):
                    acc = acc + plsc.load_gather(in_ref, [row_idx, col0 + k])
                stage[pl.ds((g * CH + r) * L, L)] = acc * (1.0 / G)

        @plsc.parallel_loop(0, P, step=1, unroll=2)
        def _tr(i):
            g = i // L
            col = i % L
            for q in range(CH // L):
                idx = lane16 + ((g * CH + q * L) * L + col)
                v = plsc.load_gather(stage, [idx])
                out_ref[i, pl.ds(half * CH + q * L, L)] = v

        if half == 1:
            out_copies[ob] = pltpu.async_copy(
                out_ref,
                out_hbm.at[:, pl.ds(row0 + (c - 1) * CH, 2 * CH)],
                osems[ob],
            )

    out_copies[0].wait()
    out_copies[1].wait()


RB = 2048


def _pool_tc_body(x_ref, o_ref):
    f = lax.broadcasted_iota(jnp.int32, (P, F), 1)
    p = lax.broadcasted_iota(jnp.int32, (P, F), 0)
    w = jnp.where(f // G == p, 1.0 / G, 0.0).astype(jnp.float32)
    o_ref[...] = lax.dot_general(
        w,
        x_ref[...],
        (((1,), (1,)), ((), ())),
        preferred_element_type=jnp.float32,
        precision=lax.Precision.HIGHEST,
    )


_pool_tc = pl.pallas_call(
    _pool_tc_body,
    grid=(B_TC // RB,),
    in_specs=[pl.BlockSpec((RB, F), lambda i: (i + B_SC // RB, 0))],
    out_specs=pl.BlockSpec((P, RB), lambda i: (0, i + B_SC // RB)),
    out_shape=jax.ShapeDtypeStruct((P, B), jnp.float32),
)


def kernel(gene_set_features):
    o_sc = _pool_sc(gene_set_features)
    o_full = _pool_tc(gene_set_features)
    return lax.dynamic_update_slice(o_full, o_sc, (0, 0)).T
